# Initial kernel scaffold; baseline (speedup 1.0000x reference)
import jax
import jax.numpy as jnp
from jax.experimental import pallas as pl


def kernel(features, Wl1, bl1, Wr1, Wl2, bl2, Wr2, Wih0, Whh0, bih0, bhh0, Wih1, Whh1, bih1, bhh1, Wfc, bfc):
    raise NotImplementedError("write your pallas kernel here")


# speedup vs baseline: 25.3068x; 25.3068x over previous
import jax
import jax.numpy as jnp
from jax.experimental import pallas as pl
from jax.experimental.pallas import tpu as pltpu


def _fused_kernel(x0_ref, wr1_ref, bl1_ref, wr2_ref, bl2_ref,
                  wih0_ref, whh0_ref, bi0_ref, bh0_ref,
                  wih1_ref, whh1_ref, bi1_ref, bh1_ref,
                  wfc_ref, bfc_ref,
                  out_ref, gi_ref, ys_ref):
    b, w, _, f = x0_ref.shape
    h = wr1_ref.shape[1]

    x0 = x0_ref[:, :, 0, :].reshape(b * w, f)
    s1 = jnp.dot(x0, wr1_ref[...], preferred_element_type=jnp.float32) + bl1_ref[...]
    seq = jnp.dot(s1, wr2_ref[...], preferred_element_type=jnp.float32) + bl2_ref[...]

    gi_ref[...] = (jnp.dot(seq, wih0_ref[...], preferred_element_type=jnp.float32)
                   + bi0_ref[...]).reshape(b, w, 3 * h)

    def make_step(whh, bhh, store):
        def step(t, hprev):
            gi = gi_ref[:, pl.ds(t, 1), :].reshape(b, 3 * h)
            gh = jnp.dot(hprev, whh, preferred_element_type=jnp.float32) + bhh
            r = jax.nn.sigmoid(gi[:, :h] + gh[:, :h])
            z = jax.nn.sigmoid(gi[:, h:2 * h] + gh[:, h:2 * h])
            n = jnp.tanh(gi[:, 2 * h:] + r * gh[:, 2 * h:])
            hnew = (1.0 - z) * n + z * hprev
            if store:
                ys_ref[:, pl.ds(t, 1), :] = hnew[:, None, :]
            return hnew
        return step

    h0 = jnp.zeros((b, h), dtype=jnp.float32)
    jax.lax.fori_loop(0, w, make_step(whh0_ref[...], bh0_ref[...], True), h0)

    gi_ref[...] = (jnp.dot(ys_ref[...].reshape(b * w, h), wih1_ref[...],
                           preferred_element_type=jnp.float32)
                   + bi1_ref[...]).reshape(b, w, 3 * h)
    h1 = jax.lax.fori_loop(0, w, make_step(whh1_ref[...], bh1_ref[...], False), h0)

    out_ref[...] = jnp.dot(h1, wfc_ref[...], preferred_element_type=jnp.float32) + bfc_ref[...]


def kernel(features, Wl1, bl1, Wr1, Wl2, bl2, Wr2, Wih0, Whh0, bih0, bhh0,
           Wih1, Whh1, bih1, bhh1, Wfc, bfc):
    b, w, s, f = features.shape
    h = Wr1.shape[0]
    horizon = Wfc.shape[0]
    hp = 128

    wfc_t = jnp.zeros((h, hp), Wfc.dtype).at[:, :horizon].set(Wfc.T)
    bfc_p = jnp.zeros((1, hp), bfc.dtype).at[0, :horizon].set(bfc)

    args = (features,
            Wr1.T, bl1.reshape(1, h), Wr2.T, bl2.reshape(1, h),
            Wih0.T, Whh0.T, bih0.reshape(1, 3 * h), bhh0.reshape(1, 3 * h),
            Wih1.T, Whh1.T, bih1.reshape(1, 3 * h), bhh1.reshape(1, 3 * h),
            wfc_t, bfc_p)

    in_specs = [pl.BlockSpec((b, w, 8, f), lambda i: (0, 0, 0, 0))]
    in_specs += [pl.BlockSpec(a.shape, (lambda nd: (lambda i: (0,) * nd))(a.ndim))
                 for a in args[1:]]

    out = pl.pallas_call(
        _fused_kernel,
        grid=(1,),
        in_specs=in_specs,
        out_specs=pl.BlockSpec((b, hp), lambda i: (0, 0)),
        out_shape=jax.ShapeDtypeStruct((b, hp), jnp.float32),
        scratch_shapes=[pltpu.VMEM((b, w, 3 * h), jnp.float32),
                        pltpu.VMEM((b, w, h), jnp.float32)],
    )(*args)
    return out[:, :horizon]


# speedup vs baseline: 33.9202x; 1.3404x over previous
import jax
import jax.numpy as jnp
from jax.experimental import pallas as pl
from jax.experimental.pallas import tpu as pltpu


def _fused_kernel(x0_ref, wr1_ref, bl1_ref, wr2_ref, bl2_ref,
                  wih0_ref, whh0_ref, bi0_ref, bh0_ref,
                  wih1_ref, whh1_ref, bi1_ref, bh1_ref,
                  wfc_ref, bfc_ref,
                  out_ref, gi_ref):
    b, w, _, f = x0_ref.shape
    h = wr1_ref.shape[1]

    x0 = jnp.transpose(x0_ref[:, :, 0, :], (1, 0, 2)).reshape(w * b, f)
    s1 = jnp.dot(x0, wr1_ref[...], preferred_element_type=jnp.float32) + bl1_ref[...]
    seq = jnp.dot(s1, wr2_ref[...], preferred_element_type=jnp.float32) + bl2_ref[...]

    gi_ref[...] = (jnp.dot(seq, wih0_ref[...], preferred_element_type=jnp.float32)
                   + bi0_ref[...]).reshape(w, b, 3 * h)

    whh0 = whh0_ref[...]
    whh1 = whh1_ref[...]
    wih1 = wih1_ref[...]
    bh0 = bh0_ref[...]
    bh1 = bh1_ref[...]
    bi1 = bi1_ref[...]

    def gates(gi, gh, hprev):
        r = jax.nn.sigmoid(gi[:, :h] + gh[:, :h])
        z = jax.nn.sigmoid(gi[:, h:2 * h] + gh[:, h:2 * h])
        n = jnp.tanh(gi[:, 2 * h:] + r * gh[:, 2 * h:])
        return (1.0 - z) * n + z * hprev

    zeros = jnp.zeros((b, h), dtype=jnp.float32)

    y_prev = gates(gi_ref[pl.ds(0, 1)].reshape(b, 3 * h), bh0, zeros)

    def step(t, carry):
        h0, h1, yp = carry
        gi0 = gi_ref[pl.ds(t, 1)].reshape(b, 3 * h)
        gh0 = jnp.dot(h0, whh0, preferred_element_type=jnp.float32) + bh0
        y_new = gates(gi0, gh0, h0)
        gi1 = jnp.dot(yp, wih1, preferred_element_type=jnp.float32) + bi1
        gh1 = jnp.dot(h1, whh1, preferred_element_type=jnp.float32) + bh1
        h1n = gates(gi1, gh1, h1)
        return y_new, h1n, y_new

    hL, h1, y_prev = jax.lax.fori_loop(1, w, step, (y_prev, zeros, y_prev))

    gi1 = jnp.dot(y_prev, wih1, preferred_element_type=jnp.float32) + bi1
    gh1 = jnp.dot(h1, whh1, preferred_element_type=jnp.float32) + bh1
    h1 = gates(gi1, gh1, h1)

    out_ref[...] = jnp.dot(h1, wfc_ref[...], preferred_element_type=jnp.float32) + bfc_ref[...]


def kernel(features, Wl1, bl1, Wr1, Wl2, bl2, Wr2, Wih0, Whh0, bih0, bhh0,
           Wih1, Whh1, bih1, bhh1, Wfc, bfc):
    b, w, s, f = features.shape
    h = Wr1.shape[0]
    horizon = Wfc.shape[0]
    hp = 128

    wfc_t = jnp.zeros((h, hp), Wfc.dtype).at[:, :horizon].set(Wfc.T)
    bfc_p = jnp.zeros((1, hp), bfc.dtype).at[0, :horizon].set(bfc)

    args = (features,
            Wr1.T, bl1.reshape(1, h), Wr2.T, bl2.reshape(1, h),
            Wih0.T, Whh0.T, bih0.reshape(1, 3 * h), bhh0.reshape(1, 3 * h),
            Wih1.T, Whh1.T, bih1.reshape(1, 3 * h), bhh1.reshape(1, 3 * h),
            wfc_t, bfc_p)

    in_specs = [pl.BlockSpec((b, w, 8, f), lambda i: (0, 0, 0, 0))]
    in_specs += [pl.BlockSpec(a.shape, (lambda nd: (lambda i: (0,) * nd))(a.ndim))
                 for a in args[1:]]

    out = pl.pallas_call(
        _fused_kernel,
        grid=(1,),
        in_specs=in_specs,
        out_specs=pl.BlockSpec((b, hp), lambda i: (0, 0)),
        out_shape=jax.ShapeDtypeStruct((b, hp), jnp.float32),
        scratch_shapes=[pltpu.VMEM((w, b, 3 * h), jnp.float32)],
    )(*args)
    return out[:, :horizon]


# speedup vs baseline: 44.2324x; 1.3040x over previous
import jax
import jax.numpy as jnp
from jax.experimental import pallas as pl
from jax.experimental.pallas import tpu as pltpu


def _dott(a, b):
    return jax.lax.dot_general(a, b, (((1,), (1,)), ((), ())),
                               preferred_element_type=jnp.float32)


def _fused_kernel(x0_ref, wr1_ref, bl1_ref, wr2_ref, bl2_ref,
                  wih0_ref, whh0_ref, bi0_ref, bh0_ref,
                  wih1_ref, whh1_ref, bi1_ref, bh1_ref,
                  wfc_ref, bfc_ref,
                  out_ref, gi_ref):
    b, w, _, f = x0_ref.shape
    h = wr1_ref.shape[1]

    x0 = jnp.transpose(x0_ref[:, :, 0, :], (1, 0, 2)).reshape(w * b, f)
    s1 = _dott(x0, wr1_ref[...]) + bl1_ref[...]
    seq = _dott(s1, wr2_ref[...]) + bl2_ref[...]

    gi_ref[...] = (_dott(seq, wih0_ref[...]) + bi0_ref[...]).reshape(w, b, 3 * h)

    whh0 = whh0_ref[...]
    whh1 = whh1_ref[...]
    wih1 = wih1_ref[...]
    bh0 = bh0_ref[...]
    bh1 = bh1_ref[...]
    bi1 = bi1_ref[...]

    def gates(gi, gh, hprev):
        r = jax.nn.sigmoid(gi[:, :h] + gh[:, :h])
        z = jax.nn.sigmoid(gi[:, h:2 * h] + gh[:, h:2 * h])
        n = jnp.tanh(gi[:, 2 * h:] + r * gh[:, 2 * h:])
        return (1.0 - z) * n + z * hprev

    zeros = jnp.zeros((b, h), dtype=jnp.float32)

    y_prev = gates(gi_ref[pl.ds(0, 1)].reshape(b, 3 * h), bh0, zeros)

    def step(t, carry):
        h0, h1, yp = carry
        gi0 = gi_ref[pl.ds(t, 1)].reshape(b, 3 * h)
        gh0 = _dott(h0, whh0) + bh0
        y_new = gates(gi0, gh0, h0)
        gi1 = _dott(yp, wih1) + bi1
        gh1 = _dott(h1, whh1) + bh1
        h1n = gates(gi1, gh1, h1)
        return y_new, h1n, y_new

    hL, h1, y_prev = jax.lax.fori_loop(1, w, step, (y_prev, zeros, y_prev))

    gi1 = _dott(y_prev, wih1) + bi1
    gh1 = _dott(h1, whh1) + bh1
    h1 = gates(gi1, gh1, h1)

    out_ref[...] = _dott(h1, wfc_ref[...]) + bfc_ref[...]


def kernel(features, Wl1, bl1, Wr1, Wl2, bl2, Wr2, Wih0, Whh0, bih0, bhh0,
           Wih1, Whh1, bih1, bhh1, Wfc, bfc):
    b, w, s, f = features.shape
    h = Wr1.shape[0]
    horizon = Wfc.shape[0]

    args = (features,
            Wr1, bl1.reshape(1, h), Wr2, bl2.reshape(1, h),
            Wih0, Whh0, bih0.reshape(1, 3 * h), bhh0.reshape(1, 3 * h),
            Wih1, Whh1, bih1.reshape(1, 3 * h), bhh1.reshape(1, 3 * h),
            Wfc, bfc.reshape(1, horizon))

    in_specs = [pl.BlockSpec((b, w, 8, f), lambda i: (0, 0, 0, 0))]
    in_specs += [pl.BlockSpec(a.shape, (lambda nd: (lambda i: (0,) * nd))(a.ndim))
                 for a in args[1:]]

    return pl.pallas_call(
        _fused_kernel,
        grid=(1,),
        in_specs=in_specs,
        out_specs=pl.BlockSpec((b, horizon), lambda i: (0, 0)),
        out_shape=jax.ShapeDtypeStruct((b, horizon), jnp.float32),
        scratch_shapes=[pltpu.VMEM((w, b, 3 * h), jnp.float32)],
    )(*args)


# speedup vs baseline: 56.3093x; 1.2730x over previous
import jax
import jax.numpy as jnp
from jax.experimental import pallas as pl
from jax.experimental.pallas import tpu as pltpu


def _dott(a, b, precision=None):
    return jax.lax.dot_general(a, b, (((1,), (1,)), ((), ())),
                               preferred_element_type=jnp.float32,
                               precision=precision)


def _fused_kernel(x0_ref, wr1_ref, bl1_ref, wr2_ref, bl2_ref,
                  wih0_ref, whh0_ref, bi0_ref, bh0_ref,
                  wih1_ref, whh1_ref, bi1_ref, bh1_ref,
                  wfc_ref, bfc_ref,
                  out_ref, gi_ref):
    b, w, _, f = x0_ref.shape
    h = wr1_ref.shape[1]

    x0 = jnp.transpose(x0_ref[:, :, 0, :], (1, 0, 2)).reshape(w * b, f)
    s1 = _dott(x0, wr1_ref[...]) + bl1_ref[...]
    seq = _dott(s1, wr2_ref[...]) + bl2_ref[...]

    gi_ref[...] = (_dott(seq, wih0_ref[...]) + bi0_ref[...]).reshape(w, b, 3 * h)

    whh1 = whh1_ref[...]
    bh0 = bh0_ref[...]
    bh1 = bh1_ref[...]
    bi1 = bi1_ref[...]
    w01 = jnp.concatenate([whh0_ref[...], wih1_ref[...]], axis=0)
    b01 = jnp.concatenate([bh0, bi1], axis=1)

    def gates(gi, gh, hprev):
        r = jax.nn.sigmoid(gi[:, :h] + gh[:, :h])
        z = jax.nn.sigmoid(gi[:, h:2 * h] + gh[:, h:2 * h])
        n = jnp.tanh(gi[:, 2 * h:] + r * gh[:, 2 * h:])
        return (1.0 - z) * n + z * hprev

    zeros = jnp.zeros((b, h), dtype=jnp.float32)

    y_prev = gates(gi_ref[pl.ds(0, 1)].reshape(b, 3 * h), bh0, zeros)

    def one_step(t, h0, h1):
        g01 = _dott(h0, w01) + b01
        gi0 = gi_ref[pl.ds(t, 1)].reshape(b, 3 * h)
        gh1 = _dott(h1, whh1) + bh1
        y_new = gates(gi0, g01[:, :3 * h], h0)
        h1n = gates(g01[:, 3 * h:], gh1, h1)
        return y_new, h1n

    unroll = 3
    def step(i, carry):
        h0, h1 = carry
        for k in range(unroll):
            h0, h1 = one_step(i * unroll + 1 + k, h0, h1)
        return h0, h1

    done = ((w - 1) // unroll) * unroll
    y_prev, h1 = jax.lax.fori_loop(0, (w - 1) // unroll, step, (y_prev, zeros))
    for k in range(w - 1 - done):
        y_prev, h1 = one_step(done + 1 + k, y_prev, h1)

    gi1 = _dott(y_prev, wih1_ref[...]) + bi1
    gh1 = _dott(h1, whh1) + bh1
    h1 = gates(gi1, gh1, h1)

    out_ref[...] = _dott(h1, wfc_ref[...]) + bfc_ref[...]


def kernel(features, Wl1, bl1, Wr1, Wl2, bl2, Wr2, Wih0, Whh0, bih0, bhh0,
           Wih1, Whh1, bih1, bhh1, Wfc, bfc):
    b, w, s, f = features.shape
    h = Wr1.shape[0]
    horizon = Wfc.shape[0]

    args = (features,
            Wr1, bl1.reshape(1, h), Wr2, bl2.reshape(1, h),
            Wih0, Whh0, bih0.reshape(1, 3 * h), bhh0.reshape(1, 3 * h),
            Wih1, Whh1, bih1.reshape(1, 3 * h), bhh1.reshape(1, 3 * h),
            Wfc, bfc.reshape(1, horizon))

    in_specs = [pl.BlockSpec((b, w, 8, f), lambda i: (0, 0, 0, 0))]
    in_specs += [pl.BlockSpec(a.shape, (lambda nd: (lambda i: (0,) * nd))(a.ndim))
                 for a in args[1:]]

    return pl.pallas_call(
        _fused_kernel,
        grid=(1,),
        in_specs=in_specs,
        out_specs=pl.BlockSpec((b, horizon), lambda i: (0, 0)),
        out_shape=jax.ShapeDtypeStruct((b, horizon), jnp.float32),
        scratch_shapes=[pltpu.VMEM((w, b, 3 * h), jnp.float32)],
    )(*args)
